# Initial kernel scaffold; baseline (speedup 1.0000x reference)
import jax
import jax.numpy as jnp
from jax.experimental import pallas as pl


def kernel(x, weight, bias):
    raise NotImplementedError("write your pallas kernel here")


# speedup vs baseline: 2.0873x; 2.0873x over previous
import functools

import jax
import jax.numpy as jnp
from jax.experimental import pallas as pl
from jax.experimental.pallas import tpu as pltpu


def _dup_tap_mats(W, rows_per_quad):
    Wd = 2 * W
    j = jnp.arange(Wd)[None, :]
    u = jnp.arange(W)[:, None]
    taps = [((j == 2 * u + 1 - kw) | (j == 2 * u + 2 - kw)).astype(jnp.float32)
            for kw in range(3)]
    d3 = jnp.stack(taps)
    eye = jnp.eye(rows_per_quad, dtype=jnp.float32)
    dq = jnp.einsum("ab,kwj->awkbj", eye, d3).reshape(
        rows_per_quad * W, 3 * rows_per_quad * Wd)
    return dq.astype(jnp.bfloat16)


def _fold_weights(weight):
    R = jnp.array([[1, 0, 0],
                   [0, 1, 1],
                   [1, 1, 0],
                   [0, 0, 1]], dtype=weight.dtype)
    w12 = jnp.einsum("gh,oihw->gwoi", R, weight)
    Co, Ci = weight.shape[0], weight.shape[1]
    w12 = w12.reshape(2, 6, Co, Ci)
    return jnp.transpose(w12, (0, 2, 1, 3)).reshape(2, Co, 6 * Ci).astype(
        jnp.bfloat16)


def _fused_kernel(xf_ref, w_ref, d_ref, b_ref, o_ref, tb_ref, sf_ref, sem,
                  *, th, W, n_chunks, n_quads, rows_per_quad):
    n = pl.program_id(0)
    nsteps = pl.num_programs(0)
    Wd = 2 * W
    Ci = xf_ref.shape[1]
    Co = sf_ref.shape[2]
    qW = rows_per_quad * W
    qWd = rows_per_quad * Wd
    bias = b_ref[...]
    slot = n % 2

    def issue(img, s):
        for co in range(Co):
            pltpu.make_async_copy(
                sf_ref.at[s, :, co, :], o_ref.at[img, co, :, :], sem).start()

    def drain(count):
        for _ in range(count):
            pltpu.make_async_copy(
                sf_ref.at[0, :, 0, :], sf_ref.at[0, :, 0, :], sem).wait()

    @pl.when(n > 1)
    def _():
        drain(Co)

    @pl.when(n > 0)
    def _():
        issue(n - 1, 1 - slot)
    for hc in range(n_chunks):
        base = hc * th * W
        for q in range(n_quads):
            xq = xf_ref[0, :, base + q * qW:base + (q + 1) * qW]
            t = jnp.dot(xq, d_ref[...],
                        preferred_element_type=jnp.float32).astype(
                            jnp.bfloat16)
            for kw in range(3):
                tkw = t[:, kw * qWd:(kw + 1) * qWd]
                tb_ref[kw * Ci:(kw + 1) * Ci, q * qWd:(q + 1) * qWd] = tkw
                if q == 0:
                    tb_ref[(3 + kw) * Ci:(4 + kw) * Ci,
                           :(rows_per_quad - 1) * Wd] = tkw[:, Wd:]
                else:
                    tb_ref[(3 + kw) * Ci:(4 + kw) * Ci,
                           (q * rows_per_quad - 1) * Wd:
                           ((q + 1) * rows_per_quad - 1) * Wd] = tkw

        for r in range(2):
            tap = tb_ref[:, r * Wd:(r + th) * Wd]
            acc = jnp.dot(w_ref[r], tap,
                          preferred_element_type=jnp.float32) + bias
            for t_ in range(th):
                i = hc * 2 * th + 2 * t_ + r
                sf_ref[slot, i] = acc[:, t_ * Wd:(t_ + 1) * Wd]

    @pl.when(n == nsteps - 1)
    def _():
        issue(n, slot)
        drain(Co if nsteps == 1 else 2 * Co)


@jax.jit
def kernel(x, weight, bias):
    N, C, H, W = x.shape
    Co, Ci = weight.shape[0], weight.shape[1]
    th = H if H % 8 == 0 else 8
    assert H % th == 0 and W % 2 == 0
    n_chunks = H // th
    Wd = 2 * W
    rows_per_quad = 4
    n_quads = (th + 2 + rows_per_quad - 1) // rows_per_quad
    pad_back = (n_chunks - 1) * th + n_quads * rows_per_quad - H - 1

    xf = jnp.pad(x.reshape(N, C, H * W),
                 ((0, 0), (0, 0), (W, pad_back * W))).astype(jnp.bfloat16)
    Sp = H * W + W + pad_back * W

    wcat = _fold_weights(weight)
    dq = _dup_tap_mats(W, rows_per_quad)
    bmat = bias.reshape(Co, 1).astype(jnp.float32)

    itemsize = jnp.dtype(x.dtype).itemsize
    cost = pl.CostEstimate(
        flops=2 * N * 2 * H * Wd * 6 * Ci * Co,
        transcendentals=0,
        bytes_accessed=(N * Ci * H * W + N * Co * 2 * H * Wd) * itemsize,
    )

    body = functools.partial(_fused_kernel, th=th, W=W, n_chunks=n_chunks,
                             n_quads=n_quads, rows_per_quad=rows_per_quad)
    return pl.pallas_call(
        body,
        out_shape=jax.ShapeDtypeStruct((N, Co, 2 * H, Wd), x.dtype),
        grid=(N,),
        in_specs=[
            pl.BlockSpec((1, Ci, Sp), lambda n: (n, 0, 0)),
            pl.BlockSpec((2, Co, 6 * Ci), lambda n: (0, 0, 0)),
            pl.BlockSpec((rows_per_quad * W, 3 * rows_per_quad * Wd),
                         lambda n: (0, 0)),
            pl.BlockSpec((Co, 1), lambda n: (0, 0)),
        ],
        out_specs=pl.BlockSpec(memory_space=pl.ANY),
        scratch_shapes=[
            pltpu.VMEM((6 * Ci, (n_quads * rows_per_quad) * Wd),
                       jnp.bfloat16),
            pltpu.VMEM((2, 2 * H, Co, Wd), jnp.float32),
            pltpu.SemaphoreType.DMA,
        ],
        compiler_params=pltpu.CompilerParams(
            dimension_semantics=("arbitrary",)),
        cost_estimate=cost,
    )(xf, wcat, dq, bmat)
